# Initial kernel scaffold; baseline (speedup 1.0000x reference)
import jax
import jax.numpy as jnp
from jax.experimental import pallas as pl


def kernel(sensor_ids, states, embeddings_tensor):
    raise NotImplementedError("write your pallas kernel here")


# speedup vs baseline: 2.8299x; 2.8299x over previous
import functools

import jax
import jax.numpy as jnp
from jax import lax
from jax.experimental import pallas as pl
from jax.experimental.pallas import tpu as pltpu
from jax.experimental.pallas import tpu_sc as plsc

_N_STATES = 10
_EMBED_DIM = 128
_TOTAL = 4096 * 50
_NC, _NS = 2, 16
_NW = _NC * _NS
_B_PER_W = _TOTAL // _NW
_G = 128
_NG = _B_PER_W // _G
_LANES = 16

_mesh = plsc.VectorSubcoreMesh(core_axis_name="c", subcore_axis_name="s")


@functools.partial(
    pl.kernel,
    mesh=_mesh,
    out_type=jax.ShapeDtypeStruct((_TOTAL, _EMBED_DIM), jnp.float32),
    scratch_types=[
        pltpu.VMEM((_B_PER_W,), jnp.int32),
        pltpu.VMEM((_B_PER_W,), jnp.int32),
        pltpu.VMEM((_NG, _G), jnp.int32),
        pltpu.VMEM((_G, _EMBED_DIM), jnp.float32),
        pltpu.VMEM((_G, _EMBED_DIM), jnp.float32),
        pltpu.SemaphoreType.DMA,
        pltpu.SemaphoreType.DMA,
    ],
)
def _lookup(sens_hbm, st_hbm, table_hbm, out_hbm,
            sens_v, st_v, idx_v, rows_a, rows_b, sem_a, sem_b):
    wid = lax.axis_index("s") * _NC + lax.axis_index("c")
    base = wid * _B_PER_W

    pltpu.sync_copy(sens_hbm.at[pl.ds(base, _B_PER_W)], sens_v)
    pltpu.sync_copy(st_hbm.at[pl.ds(base, _B_PER_W)], st_v)

    vecs_per_group = _G // _LANES

    def idx_body(i, carry):
        row = i // vecs_per_group
        col = (i % vecs_per_group) * _LANES
        s = sens_v[pl.ds(i * _LANES, _LANES)]
        t = st_v[pl.ds(i * _LANES, _LANES)]
        idx_v[row, pl.ds(col, _LANES)] = s * _N_STATES + t
        return carry

    lax.fori_loop(0, _B_PER_W // _LANES, idx_body, 0)

    def gather_body(g, carry):
        cp_a = pltpu.async_copy(table_hbm.at[idx_v.at[2 * g]], rows_a, sem_a)
        cp_b = pltpu.async_copy(table_hbm.at[idx_v.at[2 * g + 1]], rows_b, sem_b)
        cp_a.wait()
        pltpu.sync_copy(rows_a, out_hbm.at[pl.ds(base + 2 * g * _G, _G)])
        cp_b.wait()
        pltpu.sync_copy(rows_b, out_hbm.at[pl.ds(base + (2 * g + 1) * _G, _G)])
        return carry

    lax.fori_loop(0, _NG // 2, gather_body, 0)


def kernel(sensor_ids, states, embeddings_tensor):
    out = _lookup(sensor_ids.reshape(_TOTAL), states.reshape(_TOTAL),
                  embeddings_tensor)
    return out.reshape(sensor_ids.shape + (_EMBED_DIM,))


# speedup vs baseline: 2.8578x; 1.0098x over previous
import functools

import jax
import jax.numpy as jnp
from jax import lax
from jax.experimental import pallas as pl
from jax.experimental.pallas import tpu as pltpu
from jax.experimental.pallas import tpu_sc as plsc

_N_STATES = 10
_EMBED_DIM = 128
_TOTAL = 4096 * 50
_NC, _NS = 2, 16
_NW = _NC * _NS
_B_PER_W = _TOTAL // _NW
_G = 128
_NG = _B_PER_W // _G
_LANES = 16
_NBUF = 5
_LEAD = 2

_mesh = plsc.VectorSubcoreMesh(core_axis_name="c", subcore_axis_name="s")


@functools.partial(
    pl.kernel,
    mesh=_mesh,
    out_type=jax.ShapeDtypeStruct((_TOTAL, _EMBED_DIM), jnp.float32),
    scratch_types=(
        [pltpu.VMEM((_B_PER_W,), jnp.int32)] * 2 +
        [pltpu.VMEM((_NG, _G), jnp.int32)] +
        [pltpu.VMEM((_G, _EMBED_DIM), jnp.float32)] * _NBUF +
        [pltpu.SemaphoreType.DMA] * (2 * _NBUF)
    ),
)
def _lookup(sens_hbm, st_hbm, table_hbm, out_hbm, *scratch):
    sens_v, st_v, idx_v = scratch[0], scratch[1], scratch[2]
    bufs = scratch[3:3 + _NBUF]
    gsem = scratch[3 + _NBUF:3 + 2 * _NBUF]
    wsem = scratch[3 + 2 * _NBUF:3 + 3 * _NBUF]

    wid = lax.axis_index("s") * _NC + lax.axis_index("c")
    base = wid * _B_PER_W

    pltpu.sync_copy(sens_hbm.at[pl.ds(base, _B_PER_W)], sens_v)
    pltpu.sync_copy(st_hbm.at[pl.ds(base, _B_PER_W)], st_v)

    vecs_per_group = _G // _LANES

    def idx_body(i, carry):
        row = i // vecs_per_group
        col = (i % vecs_per_group) * _LANES
        s = sens_v[pl.ds(i * _LANES, _LANES)]
        t = st_v[pl.ds(i * _LANES, _LANES)]
        idx_v[row, pl.ds(col, _LANES)] = s * _N_STATES + t
        return carry

    lax.fori_loop(0, _B_PER_W // _LANES, idx_body, 0)

    def fire_gather(g, b):
        pltpu.async_copy(table_hbm.at[idx_v.at[g]], bufs[b], gsem[b])

    def wait_gather(b):
        pltpu.make_async_copy(table_hbm.at[idx_v.at[0]], bufs[b],
                              gsem[b]).wait()

    def fire_write(g, b):
        pltpu.async_copy(bufs[b], out_hbm.at[pl.ds(base + g * _G, _G)],
                         wsem[b])

    def wait_write(b):
        pltpu.make_async_copy(bufs[b], out_hbm.at[pl.ds(base, _G)],
                              wsem[b]).wait()

    for b in range(_LEAD):
        fire_gather(b, b)

    def outer(o, carry):
        for b in range(_NBUF):
            g = o * _NBUF + b
            bb = (b + _LEAD) % _NBUF

            @pl.when(jnp.logical_and(g + _LEAD >= _NBUF, g + _LEAD < _NG))
            def _():
                wait_write(bb)

            @pl.when(g + _LEAD < _NG)
            def _():
                fire_gather(g + _LEAD, bb)

            wait_gather(b)
            fire_write(g, b)
        return carry

    lax.fori_loop(0, _NG // _NBUF, outer, 0)

    for b in range(_NBUF):
        wait_write(b)


def kernel(sensor_ids, states, embeddings_tensor):
    out = _lookup(sensor_ids.reshape(_TOTAL), states.reshape(_TOTAL),
                  embeddings_tensor)
    return out.reshape(sensor_ids.shape + (_EMBED_DIM,))


# speedup vs baseline: 4.2619x; 1.4913x over previous
import functools

import jax
import jax.numpy as jnp
from jax import lax
from jax.experimental import pallas as pl
from jax.experimental.pallas import tpu as pltpu
from jax.experimental.pallas import tpu_sc as plsc

_N_STATES = 10
_TABLE_ROWS = 1000
_EMBED_DIM = 128
_BATCH = 4096
_SEQ = 50
_SEQ_PAD = 56
_TOTAL = _BATCH * _SEQ
_NC, _NS = 2, 16
_NW = _NC * _NS
_B_PER_W = _BATCH // _NW
_F_PER_W = _B_PER_W * _SEQ
_LANES = 16
_NBUF = 8
_LEAD = 4

_mesh = plsc.VectorSubcoreMesh(core_axis_name="c", subcore_axis_name="s")


@functools.partial(
    pl.kernel,
    mesh=_mesh,
    out_type=jax.ShapeDtypeStruct((_BATCH, _SEQ, _EMBED_DIM), jnp.float32),
    scratch_types=(
        [pltpu.VMEM((_F_PER_W + 32,), jnp.int32)] * 2 +
        [pltpu.VMEM((_B_PER_W, _SEQ_PAD), jnp.int32)] +
        [pltpu.VMEM((_SEQ_PAD, _EMBED_DIM), jnp.float32)] * _NBUF +
        [pltpu.SemaphoreType.DMA] * (2 * _NBUF)
    ),
)
def _lookup(sens_hbm, st_hbm, table_hbm, out_hbm, *scratch):
    sens_v, st_v, idx_v = scratch[0], scratch[1], scratch[2]
    bufs = scratch[3:3 + _NBUF]
    gsem = scratch[3 + _NBUF:3 + 2 * _NBUF]
    wsem = scratch[3 + 2 * _NBUF:3 + 3 * _NBUF]

    wid = lax.axis_index("s") * _NC + lax.axis_index("c")
    fbase = wid * _F_PER_W
    bbase = wid * _B_PER_W

    pltpu.sync_copy(sens_hbm.at[pl.ds(fbase, _F_PER_W)],
                    sens_v.at[pl.ds(0, _F_PER_W)])
    pltpu.sync_copy(st_hbm.at[pl.ds(fbase, _F_PER_W)],
                    st_v.at[pl.ds(0, _F_PER_W)])

    def idx_body(g, carry):
        for col in (0, 16, 32, 40):
            s = sens_v[pl.ds(g * _SEQ + col, _LANES)]
            t = st_v[pl.ds(g * _SEQ + col, _LANES)]
            flat = jnp.clip(s * _N_STATES + t, 0, _TABLE_ROWS - 1)
            idx_v[g, pl.ds(col, _LANES)] = flat
        return carry

    lax.fori_loop(0, _B_PER_W, idx_body, 0)

    def fire_gather(g, b):
        pltpu.async_copy(table_hbm.at[idx_v.at[g]], bufs[b], gsem[b])

    def wait_gather(b):
        pltpu.make_async_copy(table_hbm.at[idx_v.at[0]], bufs[b],
                              gsem[b]).wait()

    def fire_write(g, b):
        pltpu.async_copy(bufs[b].at[pl.ds(0, _SEQ)], out_hbm.at[bbase + g],
                         wsem[b])

    def wait_write(b):
        pltpu.make_async_copy(bufs[b].at[pl.ds(0, _SEQ)], out_hbm.at[bbase],
                              wsem[b]).wait()

    for b in range(_LEAD):
        fire_gather(b, b)

    def outer(o, carry):
        for b in range(_NBUF):
            g = o * _NBUF + b
            bb = (b + _LEAD) % _NBUF

            @pl.when(jnp.logical_and(g + _LEAD >= _NBUF,
                                     g + _LEAD < _B_PER_W))
            def _():
                wait_write(bb)

            @pl.when(g + _LEAD < _B_PER_W)
            def _():
                fire_gather(g + _LEAD, bb)

            wait_gather(b)
            fire_write(g, b)
        return carry

    lax.fori_loop(0, _B_PER_W // _NBUF, outer, 0)

    for b in range(_NBUF):
        wait_write(b)


def kernel(sensor_ids, states, embeddings_tensor):
    return _lookup(sensor_ids.reshape(_TOTAL), states.reshape(_TOTAL),
                   embeddings_tensor)


# speedup vs baseline: 15.8995x; 3.7306x over previous
import functools

import jax
import jax.numpy as jnp
from jax import lax
from jax.experimental import pallas as pl
from jax.experimental.pallas import tpu as pltpu
from jax.experimental.pallas import tpu_sc as plsc

_N_STATES = 10
_EMBED_DIM = 128
_BATCH = 4096
_SEQ = 50
_NC, _NS = 2, 16
_NW = _NC * _NS
_B_PER_W = _BATCH // _NW
_LANES = 16
_NBUF = 5
_LEAD = 2

_mesh = plsc.VectorSubcoreMesh(core_axis_name="c", subcore_axis_name="s")


@functools.partial(
    pl.kernel,
    mesh=_mesh,
    out_type=jax.ShapeDtypeStruct((_SEQ, _BATCH, _EMBED_DIM), jnp.float32),
    scratch_types=(
        [pltpu.VMEM((_SEQ, _B_PER_W), jnp.int32)] * 3 +
        [pltpu.VMEM((_B_PER_W, _EMBED_DIM), jnp.float32)] * _NBUF +
        [pltpu.SemaphoreType.DMA] * (2 * _NBUF) +
        [pltpu.VMEM_SHARED((1000, _EMBED_DIM), jnp.float32)] +
        [pltpu.SemaphoreType.DMA] * 2
    ),
)
def _lookup(sens_hbm, st_hbm, table_hbm, out_hbm, *scratch):
    sens_v, st_v, idx_v = scratch[0], scratch[1], scratch[2]
    bufs = scratch[3:3 + _NBUF]
    gsem = scratch[3 + _NBUF:3 + 2 * _NBUF]
    wsem = scratch[3 + 2 * _NBUF:3 + 3 * _NBUF]
    table_sp = scratch[3 + 3 * _NBUF]
    isem_a, isem_b = scratch[4 + 3 * _NBUF], scratch[5 + 3 * _NBUF]

    sid = lax.axis_index("s")
    wid = sid * _NC + lax.axis_index("c")
    bbase = wid * _B_PER_W

    in_a = pltpu.async_copy(sens_hbm.at[:, pl.ds(bbase, _B_PER_W)], sens_v,
                            isem_a)
    in_b = pltpu.async_copy(st_hbm.at[:, pl.ds(bbase, _B_PER_W)], st_v,
                            isem_b)

    @pl.when(sid == 0)
    def _():
        pltpu.sync_copy(table_hbm, table_sp)

    vecs_per_row = _B_PER_W // _LANES

    def compute_idx_row(r):
        def body(j, carry):
            col = j * _LANES
            s = sens_v[r, pl.ds(col, _LANES)]
            t = st_v[r, pl.ds(col, _LANES)]
            idx_v[r, pl.ds(col, _LANES)] = s * _N_STATES + t
            return carry
        lax.fori_loop(0, vecs_per_row, body, 0)

    def fire_gather(g, b, src):
        pltpu.async_copy(src.at[idx_v.at[g]], bufs[b], gsem[b])

    def wait_gather(b):
        pltpu.make_async_copy(table_sp.at[idx_v.at[0]], bufs[b],
                              gsem[b]).wait()

    def fire_write(g, b):
        pltpu.async_copy(bufs[b], out_hbm.at[g, pl.ds(bbase, _B_PER_W)],
                         wsem[b])

    def wait_write(b):
        pltpu.make_async_copy(bufs[b], out_hbm.at[0, pl.ds(bbase, _B_PER_W)],
                              wsem[b]).wait()

    plsc.subcore_barrier()
    in_a.wait()
    in_b.wait()

    def src_for(b):
        return table_sp

    for b in range(_LEAD):
        compute_idx_row(b)
        fire_gather(b, b, src_for(b))

    def outer(o, carry):
        for b in range(_NBUF):
            g = o * _NBUF + b
            bb = (b + _LEAD) % _NBUF

            @pl.when(jnp.logical_and(g + _LEAD >= _NBUF, g + _LEAD < _SEQ))
            def _():
                wait_write(bb)

            @pl.when(g + _LEAD < _SEQ)
            def _():
                compute_idx_row(g + _LEAD)
                fire_gather(g + _LEAD, bb, src_for(bb))

            wait_gather(b)
            fire_write(g, b)
        return carry

    lax.fori_loop(0, _SEQ // _NBUF, outer, 0)

    for b in range(_NBUF):
        wait_write(b)


def kernel(sensor_ids, states, embeddings_tensor):
    out_t = _lookup(sensor_ids.T, states.T, embeddings_tensor)
    return out_t.transpose(1, 0, 2)
